# baseline (speedup 1.0000x reference)
import functools

import jax
import jax.numpy as jnp
from jax import lax
from jax.experimental import pallas as pl
from jax.experimental.pallas import tpu as pltpu
from jax.experimental.pallas import tpu_sc as plsc

_NUM_CLASSES = 80
_THRESH = 0.25
_STRIDES = (8.0, 16.0, 32.0)

_N = 8400
_LANES = 16
_CHUNK = _N // _LANES
_BINS = 512
_MASK = _BINS - 1
_INVALID_KEY = 1 << 24
_GCHUNKS = 66
_NPAD = _GCHUNKS * 128


def _decode_one(f, stride):
    c, h, w = f.shape
    gx = jax.lax.broadcasted_iota(jnp.int32, (1, h, w), 2).astype(jnp.float32)
    gy = jax.lax.broadcasted_iota(jnp.int32, (1, h, w), 1).astype(jnp.float32)
    bx = (f[0:1] + gx) * stride
    by = (f[1:2] + gy) * stride
    bw = jnp.exp(f[2:3]) * stride
    bh = jnp.exp(f[3:4]) * stride
    x1 = bx - bw / 2.0
    y1 = by - bh / 2.0
    x2 = bx + bw / 2.0
    y2 = by + bh / 2.0
    obj = jax.nn.sigmoid(f[4:5])
    prod = jax.nn.sigmoid(f[5:5 + _NUM_CLASSES]) * obj
    m = jnp.max(prod, axis=0, keepdims=True)
    ids = jax.lax.broadcasted_iota(jnp.int32, prod.shape, 0)
    cid = jnp.min(jnp.where(prod == m, ids, _NUM_CLASSES), axis=0,
                  keepdims=True)
    masked = jnp.where(m > _THRESH, m, -1.0)
    return masked, cid, x1, y1, x2, y2


def _decode_body(f8_ref, f16_ref, f32_ref, *out_refs):
    for lvl, (ref, stride) in enumerate(((f8_ref, 8.0), (f16_ref, 16.0),
                                         (f32_ref, 32.0))):
        outs = _decode_one(ref[0], stride)
        for oix in range(6):
            out_refs[oix * 3 + lvl][0] = outs[oix]


def _decode(feat_s8, feat_s16, feat_s32):
    b = feat_s8.shape[0]
    hws = ((80, 80), (40, 40), (20, 20))
    dts = (jnp.float32, jnp.int32) + (jnp.float32,) * 4
    outs = pl.pallas_call(
        _decode_body,
        grid=(b,),
        in_specs=[
            pl.BlockSpec((1, 85, 80, 80), lambda i: (i, 0, 0, 0)),
            pl.BlockSpec((1, 85, 40, 40), lambda i: (i, 0, 0, 0)),
            pl.BlockSpec((1, 85, 20, 20), lambda i: (i, 0, 0, 0)),
        ],
        out_specs=[pl.BlockSpec((1, 1, h, w), lambda i: (i, 0, 0, 0))
                   for _ in dts for (h, w) in hws],
        out_shape=[jax.ShapeDtypeStruct((b, 1, h, w), dt)
                   for dt in dts for (h, w) in hws],
    )(feat_s8, feat_s16, feat_s32)
    return [jnp.concatenate([outs[oix * 3 + lvl].reshape(b, -1)
                             for lvl in range(3)], axis=1)
            for oix in range(6)]


def _radix_pass(shift, lane, ones, key_src, idx_src, key_dst, idx_dst, hist):
    zeros = jnp.zeros((_LANES,), jnp.int32)

    def zero_body(b, carry):
        hist[pl.ds(b * 16, 16)] = zeros
        return carry

    lax.fori_loop(0, _BINS, zero_body, 0)

    def hist_body(i, carry):
        iv = lane * _CHUNK + i
        k = plsc.load_gather(key_src, [iv])
        d = (k >> shift) & _MASK
        plsc.addupdate_scatter(hist, [d * 16 + lane], ones)
        return carry

    lax.fori_loop(0, _CHUNK, hist_body, 0)

    def scan_body(b, carry):
        v = hist[pl.ds(b * 16, 16)]
        inc = plsc.cumsum(v)
        hist[pl.ds(b * 16, 16)] = inc - v + carry
        return carry + jnp.sum(v, axis=0)

    lax.fori_loop(0, _BINS, scan_body, jnp.int32(0))


def _radix_permute(shift, lane, key_src, idx_src, key_dst, idx_dst, hist):
    def perm_body(i, carry):
        iv = lane * _CHUNK + i
        k = plsc.load_gather(key_src, [iv])
        x = plsc.load_gather(idx_src, [iv])
        d = (k >> shift) & _MASK
        h = d * 16 + lane
        pos = plsc.load_gather(hist, [h])
        plsc.store_scatter(key_dst, [pos], k)
        plsc.store_scatter(idx_dst, [pos], x)
        plsc.store_scatter(hist, [h], pos + 1)
        return carry

    lax.fori_loop(0, _CHUNK, perm_body, 0)


def _sc_sort_body(score_hbm, cls_hbm, b0_hbm, b1_hbm, b2_hbm, b3_hbm,
                  out_score, out_cls, ob0, ob1, ob2, ob3, out_cnt,
                  score_v, cls_v, key_a, key_b, idx_a, idx_b, hist,
                  p0, p1, p2, p3, pout, cnt_v):
    nc = 2
    wid = lax.axis_index("s") * nc + lax.axis_index("c")
    lane = lax.iota(jnp.int32, 16)
    ones = jnp.ones((_LANES,), jnp.int32)

    @pl.when(wid < 16)
    def _():
        r = wid
        pltpu.sync_copy(score_hbm.at[r], score_v)
        pltpu.sync_copy(cls_hbm.at[r], cls_v)

        def init_body(i, carry):
            s = score_v[pl.ds(i * 16, 16)]
            b = lax.bitcast_convert_type(s, jnp.int32)
            k = jnp.where(s > 0.0, 0x3F800000 - b, _INVALID_KEY)
            key_a[pl.ds(i * 16, 16)] = k
            idx_a[pl.ds(i * 16, 16)] = lane + i * 16
            return carry

        lax.fori_loop(0, _CHUNK, init_body, 0)

        for shift, src_k, src_i, dst_k, dst_i in (
                (0, key_a, idx_a, key_b, idx_b),
                (9, key_b, idx_b, key_a, idx_a),
                (18, key_a, idx_a, key_b, idx_b)):
            _radix_pass(shift, lane, ones, src_k, src_i, dst_k, dst_i, hist)
            if shift == 18:
                cnt_v[pl.ds(0, 16)] = hist[pl.ds(64 * 16, 16)]
            _radix_permute(shift, lane, src_k, src_i, dst_k, dst_i, hist)

        def sout_body(i, carry):
            k = key_b[pl.ds(i * 16, 16)]
            s = lax.bitcast_convert_type(0x3F800000 - k, jnp.float32)
            score_v[pl.ds(i * 16, 16)] = jnp.where(k < _INVALID_KEY, s, 0.0)
            return carry

        lax.fori_loop(0, _CHUNK, sout_body, 0)

        def gout_body(i, carry):
            x = idx_b[pl.ds(i * 16, 16)]
            key_a[pl.ds(i * 16, 16)] = plsc.load_gather(cls_v, [x])
            return carry

        lax.fori_loop(0, _CHUNK, gout_body, 0)

        pltpu.sync_copy(score_v, out_score.at[r])
        pltpu.sync_copy(key_a, out_cls.at[r])
        pltpu.sync_copy(cnt_v, out_cnt.at[r])

        for src_hbm, plane in ((b0_hbm, p0), (b1_hbm, p1),
                               (b2_hbm, p2), (b3_hbm, p3)):
            pltpu.sync_copy(src_hbm.at[r], plane)
        for plane, dst_hbm in ((p0, ob0), (p1, ob1), (p2, ob2), (p3, ob3)):
            def box_body(i, carry):
                x = idx_b[pl.ds(i * 16, 16)]
                pout[pl.ds(i * 16, 16)] = plsc.load_gather(plane, [x])
                return carry

            lax.fori_loop(0, _CHUNK, box_body, 0)
            pltpu.sync_copy(pout, dst_hbm.at[r])


def _sc_sort(scores, clss, planes):
    mesh = plsc.VectorSubcoreMesh(core_axis_name="c", subcore_axis_name="s",
                                  num_cores=2, num_subcores=16)
    return pl.kernel(
        _sc_sort_body,
        out_type=[
            jax.ShapeDtypeStruct((16, _N), jnp.float32),
            jax.ShapeDtypeStruct((16, _N), jnp.int32),
            jax.ShapeDtypeStruct((16, _N), jnp.float32),
            jax.ShapeDtypeStruct((16, _N), jnp.float32),
            jax.ShapeDtypeStruct((16, _N), jnp.float32),
            jax.ShapeDtypeStruct((16, _N), jnp.float32),
            jax.ShapeDtypeStruct((16, 16), jnp.int32),
        ],
        mesh=mesh,
        compiler_params=pltpu.CompilerParams(needs_layout_passes=False,
                                             use_tc_tiling_on_sc=False),
        scratch_types=[
            pltpu.VMEM((_N,), jnp.float32),
            pltpu.VMEM((_N,), jnp.int32),
            pltpu.VMEM((_N,), jnp.int32),
            pltpu.VMEM((_N,), jnp.int32),
            pltpu.VMEM((_N,), jnp.int32),
            pltpu.VMEM((_N,), jnp.int32),
            pltpu.VMEM((_BINS * 16,), jnp.int32),
            pltpu.VMEM((_N,), jnp.float32),
            pltpu.VMEM((_N,), jnp.float32),
            pltpu.VMEM((_N,), jnp.float32),
            pltpu.VMEM((_N,), jnp.float32),
            pltpu.VMEM((_N,), jnp.float32),
            pltpu.VMEM((16,), jnp.int32),
        ],
    )(scores, clss, *planes)


def kernel(feat_s8, feat_s16, feat_s32):
    scores, clss, *planes = _decode(feat_s8, feat_s16, feat_s32)
    out_score, out_cls, b0, b1, b2, b3, out_cnt = _sc_sort(scores, clss,
                                                           planes)
    out_boxes = jnp.stack([b0, b1, b2, b3], axis=2)
    return (out_boxes, out_score, out_cls.astype(jnp.int64), out_cnt[:, 0])


# speedup vs baseline: 1.0250x; 1.0250x over previous
import functools

import jax
import jax.numpy as jnp
from jax import lax
from jax.experimental import pallas as pl
from jax.experimental.pallas import tpu as pltpu
from jax.experimental.pallas import tpu_sc as plsc

_NUM_CLASSES = 80
_THRESH = 0.25
_STRIDES = (8.0, 16.0, 32.0)

_N = 8400
_LANES = 16
_CHUNK = _N // _LANES
_BINS = 512
_MASK = _BINS - 1
_INVALID_KEY = 1 << 24
_GCHUNKS = 66
_NPAD = _GCHUNKS * 128


def _decode_one(f, stride):
    c, h, w = f.shape
    gx = jax.lax.broadcasted_iota(jnp.int32, (1, h, w), 2).astype(jnp.float32)
    gy = jax.lax.broadcasted_iota(jnp.int32, (1, h, w), 1).astype(jnp.float32)
    bx = (f[0:1] + gx) * stride
    by = (f[1:2] + gy) * stride
    bw = jnp.exp(f[2:3]) * stride
    bh = jnp.exp(f[3:4]) * stride
    x1 = bx - bw / 2.0
    y1 = by - bh / 2.0
    x2 = bx + bw / 2.0
    y2 = by + bh / 2.0
    obj = jax.nn.sigmoid(f[4:5])
    prod = jax.nn.sigmoid(f[5:5 + _NUM_CLASSES]) * obj
    m = jnp.max(prod, axis=0, keepdims=True)
    ids = jax.lax.broadcasted_iota(jnp.int32, prod.shape, 0)
    cid = jnp.min(jnp.where(prod == m, ids, _NUM_CLASSES), axis=0,
                  keepdims=True)
    masked = jnp.where(m > _THRESH, m, -1.0)
    return masked, cid, x1, y1, x2, y2


def _decode_body(f8_ref, f16_ref, f32_ref, *out_refs):
    for lvl, (ref, stride) in enumerate(((f8_ref, 8.0), (f16_ref, 16.0),
                                         (f32_ref, 32.0))):
        outs = _decode_one(ref[0], stride)
        for oix in range(6):
            out_refs[oix * 3 + lvl][0] = outs[oix]


def _decode(feat_s8, feat_s16, feat_s32):
    b = feat_s8.shape[0]
    hws = ((80, 80), (40, 40), (20, 20))
    dts = (jnp.float32, jnp.int32) + (jnp.float32,) * 4
    outs = pl.pallas_call(
        _decode_body,
        grid=(b,),
        in_specs=[
            pl.BlockSpec((1, 85, 80, 80), lambda i: (i, 0, 0, 0)),
            pl.BlockSpec((1, 85, 40, 40), lambda i: (i, 0, 0, 0)),
            pl.BlockSpec((1, 85, 20, 20), lambda i: (i, 0, 0, 0)),
        ],
        out_specs=[pl.BlockSpec((1, 1, h, w), lambda i: (i, 0, 0, 0))
                   for _ in dts for (h, w) in hws],
        out_shape=[jax.ShapeDtypeStruct((b, 1, h, w), dt)
                   for dt in dts for (h, w) in hws],
    )(feat_s8, feat_s16, feat_s32)
    return [jnp.concatenate([outs[oix * 3 + lvl].reshape(b, -1)
                             for lvl in range(3)], axis=1)
            for oix in range(6)]


def _radix_pass(shift, lane, ones, key_src, idx_src, key_dst, idx_dst, hist):
    zeros = jnp.zeros((_LANES,), jnp.int32)

    def zero_body(b, carry):
        hist[pl.ds(b * 16, 16)] = zeros
        return carry

    lax.fori_loop(0, _BINS, zero_body, 0, unroll=8)

    def hist_body(i, carry):
        iv = lane * _CHUNK + i
        k = plsc.load_gather(key_src, [iv])
        d = (k >> shift) & _MASK
        plsc.addupdate_scatter(hist, [d * 16 + lane], ones)
        return carry

    lax.fori_loop(0, _CHUNK, hist_body, 0, unroll=5)

    def scan_body(b, carry):
        v = hist[pl.ds(b * 16, 16)]
        inc = plsc.cumsum(v)
        hist[pl.ds(b * 16, 16)] = inc - v + carry
        return carry + jnp.sum(v, axis=0)

    lax.fori_loop(0, _BINS, scan_body, jnp.int32(0), unroll=4)


def _radix_permute(shift, lane, key_src, idx_src, key_dst, idx_dst, hist):
    def perm_body(i, carry):
        iv = lane * _CHUNK + i
        k = plsc.load_gather(key_src, [iv])
        x = plsc.load_gather(idx_src, [iv])
        d = (k >> shift) & _MASK
        h = d * 16 + lane
        pos = plsc.load_gather(hist, [h])
        plsc.store_scatter(key_dst, [pos], k)
        plsc.store_scatter(idx_dst, [pos], x)
        plsc.store_scatter(hist, [h], pos + 1)
        return carry

    lax.fori_loop(0, _CHUNK, perm_body, 0, unroll=3)


def _sc_sort_body(score_hbm, cls_hbm, b0_hbm, b1_hbm, b2_hbm, b3_hbm,
                  out_score, out_cls, ob0, ob1, ob2, ob3, out_cnt,
                  score_v, cls_v, key_a, key_b, idx_a, idx_b, hist,
                  p0, p1, p2, p3, pout, cnt_v):
    nc = 2
    wid = lax.axis_index("s") * nc + lax.axis_index("c")
    lane = lax.iota(jnp.int32, 16)
    ones = jnp.ones((_LANES,), jnp.int32)

    @pl.when(wid < 16)
    def _():
        r = wid
        pltpu.sync_copy(score_hbm.at[r], score_v)
        pltpu.sync_copy(cls_hbm.at[r], cls_v)

        def init_body(i, carry):
            s = score_v[pl.ds(i * 16, 16)]
            b = lax.bitcast_convert_type(s, jnp.int32)
            k = jnp.where(s > 0.0, 0x3F800000 - b, _INVALID_KEY)
            key_a[pl.ds(i * 16, 16)] = k
            idx_a[pl.ds(i * 16, 16)] = lane + i * 16
            return carry

        lax.fori_loop(0, _CHUNK, init_body, 0, unroll=5)

        for shift, src_k, src_i, dst_k, dst_i in (
                (0, key_a, idx_a, key_b, idx_b),
                (9, key_b, idx_b, key_a, idx_a),
                (18, key_a, idx_a, key_b, idx_b)):
            _radix_pass(shift, lane, ones, src_k, src_i, dst_k, dst_i, hist)
            if shift == 18:
                cnt_v[pl.ds(0, 16)] = hist[pl.ds(64 * 16, 16)]
            _radix_permute(shift, lane, src_k, src_i, dst_k, dst_i, hist)

        def sout_body(i, carry):
            k = key_b[pl.ds(i * 16, 16)]
            s = lax.bitcast_convert_type(0x3F800000 - k, jnp.float32)
            score_v[pl.ds(i * 16, 16)] = jnp.where(k < _INVALID_KEY, s, 0.0)
            return carry

        lax.fori_loop(0, _CHUNK, sout_body, 0, unroll=5)

        def gout_body(i, carry):
            x = idx_b[pl.ds(i * 16, 16)]
            key_a[pl.ds(i * 16, 16)] = plsc.load_gather(cls_v, [x])
            return carry

        lax.fori_loop(0, _CHUNK, gout_body, 0, unroll=5)

        pltpu.sync_copy(score_v, out_score.at[r])
        pltpu.sync_copy(key_a, out_cls.at[r])
        pltpu.sync_copy(cnt_v, out_cnt.at[r])

        for src_hbm, plane in ((b0_hbm, p0), (b1_hbm, p1),
                               (b2_hbm, p2), (b3_hbm, p3)):
            pltpu.sync_copy(src_hbm.at[r], plane)
        for plane, dst_hbm in ((p0, ob0), (p1, ob1), (p2, ob2), (p3, ob3)):
            def box_body(i, carry):
                x = idx_b[pl.ds(i * 16, 16)]
                pout[pl.ds(i * 16, 16)] = plsc.load_gather(plane, [x])
                return carry

            lax.fori_loop(0, _CHUNK, box_body, 0, unroll=5)
            pltpu.sync_copy(pout, dst_hbm.at[r])


def _sc_sort(scores, clss, planes):
    mesh = plsc.VectorSubcoreMesh(core_axis_name="c", subcore_axis_name="s",
                                  num_cores=2, num_subcores=16)
    return pl.kernel(
        _sc_sort_body,
        out_type=[
            jax.ShapeDtypeStruct((16, _N), jnp.float32),
            jax.ShapeDtypeStruct((16, _N), jnp.int32),
            jax.ShapeDtypeStruct((16, _N), jnp.float32),
            jax.ShapeDtypeStruct((16, _N), jnp.float32),
            jax.ShapeDtypeStruct((16, _N), jnp.float32),
            jax.ShapeDtypeStruct((16, _N), jnp.float32),
            jax.ShapeDtypeStruct((16, 16), jnp.int32),
        ],
        mesh=mesh,
        compiler_params=pltpu.CompilerParams(needs_layout_passes=False,
                                             use_tc_tiling_on_sc=False),
        scratch_types=[
            pltpu.VMEM((_N,), jnp.float32),
            pltpu.VMEM((_N,), jnp.int32),
            pltpu.VMEM((_N,), jnp.int32),
            pltpu.VMEM((_N,), jnp.int32),
            pltpu.VMEM((_N,), jnp.int32),
            pltpu.VMEM((_N,), jnp.int32),
            pltpu.VMEM((_BINS * 16,), jnp.int32),
            pltpu.VMEM((_N,), jnp.float32),
            pltpu.VMEM((_N,), jnp.float32),
            pltpu.VMEM((_N,), jnp.float32),
            pltpu.VMEM((_N,), jnp.float32),
            pltpu.VMEM((_N,), jnp.float32),
            pltpu.VMEM((16,), jnp.int32),
        ],
    )(scores, clss, *planes)


def kernel(feat_s8, feat_s16, feat_s32):
    scores, clss, *planes = _decode(feat_s8, feat_s16, feat_s32)
    out_score, out_cls, b0, b1, b2, b3, out_cnt = _sc_sort(scores, clss,
                                                           planes)
    out_boxes = jnp.stack([b0, b1, b2, b3], axis=2)
    return (out_boxes, out_score, out_cls.astype(jnp.int64), out_cnt[:, 0])


# speedup vs baseline: 1.0590x; 1.0331x over previous
import functools

import jax
import jax.numpy as jnp
from jax import lax
from jax.experimental import pallas as pl
from jax.experimental.pallas import tpu as pltpu
from jax.experimental.pallas import tpu_sc as plsc

_NUM_CLASSES = 80
_THRESH = 0.25
_STRIDES = (8.0, 16.0, 32.0)

_N = 8400
_LANES = 16
_CHUNK = _N // _LANES
_BINS = 512
_MASK = _BINS - 1
_INVALID_KEY = 1 << 24
_GCHUNKS = 66
_NPAD = _GCHUNKS * 128


def _decode_one(f, stride):
    c, h, w = f.shape
    gx = jax.lax.broadcasted_iota(jnp.int32, (1, h, w), 2).astype(jnp.float32)
    gy = jax.lax.broadcasted_iota(jnp.int32, (1, h, w), 1).astype(jnp.float32)
    bx = (f[0:1] + gx) * stride
    by = (f[1:2] + gy) * stride
    bw = jnp.exp(f[2:3]) * stride
    bh = jnp.exp(f[3:4]) * stride
    x1 = bx - bw / 2.0
    y1 = by - bh / 2.0
    x2 = bx + bw / 2.0
    y2 = by + bh / 2.0
    obj = jax.nn.sigmoid(f[4:5])
    logits = f[5:5 + _NUM_CLASSES]
    lmax = jnp.max(logits, axis=0, keepdims=True)
    m = jax.nn.sigmoid(lmax) * obj
    ids = jax.lax.broadcasted_iota(jnp.int32, logits.shape, 0)
    cid = jnp.min(jnp.where(logits == lmax, ids, _NUM_CLASSES), axis=0,
                  keepdims=True)
    masked = jnp.where(m > _THRESH, m, -1.0)
    return masked, cid, x1, y1, x2, y2


def _decode_body(f8_ref, f16_ref, f32_ref, *out_refs):
    for lvl, (ref, stride) in enumerate(((f8_ref, 8.0), (f16_ref, 16.0),
                                         (f32_ref, 32.0))):
        outs = _decode_one(ref[0], stride)
        for oix in range(6):
            out_refs[oix * 3 + lvl][0] = outs[oix]


def _decode(feat_s8, feat_s16, feat_s32):
    b = feat_s8.shape[0]
    hws = ((80, 80), (40, 40), (20, 20))
    dts = (jnp.float32, jnp.int32) + (jnp.float32,) * 4
    outs = pl.pallas_call(
        _decode_body,
        grid=(b,),
        in_specs=[
            pl.BlockSpec((1, 85, 80, 80), lambda i: (i, 0, 0, 0)),
            pl.BlockSpec((1, 85, 40, 40), lambda i: (i, 0, 0, 0)),
            pl.BlockSpec((1, 85, 20, 20), lambda i: (i, 0, 0, 0)),
        ],
        out_specs=[pl.BlockSpec((1, 1, h, w), lambda i: (i, 0, 0, 0))
                   for _ in dts for (h, w) in hws],
        out_shape=[jax.ShapeDtypeStruct((b, 1, h, w), dt)
                   for dt in dts for (h, w) in hws],
    )(feat_s8, feat_s16, feat_s32)
    return [jnp.concatenate([outs[oix * 3 + lvl].reshape(b, -1)
                             for lvl in range(3)], axis=1)
            for oix in range(6)]


def _radix_pass(shift, lane, ones, key_src, idx_src, key_dst, idx_dst, hist):
    zeros = jnp.zeros((_LANES,), jnp.int32)

    def zero_body(b, carry):
        hist[pl.ds(b * 16, 16)] = zeros
        return carry

    lax.fori_loop(0, _BINS, zero_body, 0, unroll=8)

    def hist_body(i, carry):
        iv = lane * _CHUNK + i
        k = plsc.load_gather(key_src, [iv])
        d = (k >> shift) & _MASK
        plsc.addupdate_scatter(hist, [d * 16 + lane], ones)
        return carry

    lax.fori_loop(0, _CHUNK, hist_body, 0, unroll=5)

    def scan_body(b, carry):
        v = hist[pl.ds(b * 16, 16)]
        inc = plsc.cumsum(v)
        hist[pl.ds(b * 16, 16)] = inc - v + carry
        return carry + jnp.sum(v, axis=0)

    lax.fori_loop(0, _BINS, scan_body, jnp.int32(0), unroll=4)


def _radix_permute(shift, lane, key_src, idx_src, key_dst, idx_dst, hist):
    def perm_body(i, carry):
        iv = lane * _CHUNK + i
        k = plsc.load_gather(key_src, [iv])
        x = plsc.load_gather(idx_src, [iv])
        d = (k >> shift) & _MASK
        h = d * 16 + lane
        pos = plsc.load_gather(hist, [h])
        plsc.store_scatter(key_dst, [pos], k)
        plsc.store_scatter(idx_dst, [pos], x)
        plsc.store_scatter(hist, [h], pos + 1)
        return carry

    lax.fori_loop(0, _CHUNK, perm_body, 0, unroll=3)


def _sc_sort_body(score_hbm, cls_hbm, b0_hbm, b1_hbm, b2_hbm, b3_hbm,
                  out_score, out_cls, ob0, ob1, ob2, ob3, out_cnt,
                  score_v, cls_v, key_a, key_b, idx_a, idx_b, hist,
                  p0, p1, p2, p3, pout, cnt_v):
    nc = 2
    wid = lax.axis_index("s") * nc + lax.axis_index("c")
    lane = lax.iota(jnp.int32, 16)
    ones = jnp.ones((_LANES,), jnp.int32)

    @pl.when(wid < 16)
    def _():
        r = wid
        pltpu.sync_copy(score_hbm.at[r], score_v)
        pltpu.sync_copy(cls_hbm.at[r], cls_v)

        def init_body(i, carry):
            s = score_v[pl.ds(i * 16, 16)]
            b = lax.bitcast_convert_type(s, jnp.int32)
            k = jnp.where(s > 0.0, 0x3F800000 - b, _INVALID_KEY)
            key_a[pl.ds(i * 16, 16)] = k
            idx_a[pl.ds(i * 16, 16)] = lane + i * 16
            return carry

        lax.fori_loop(0, _CHUNK, init_body, 0, unroll=5)

        for shift, src_k, src_i, dst_k, dst_i in (
                (0, key_a, idx_a, key_b, idx_b),
                (9, key_b, idx_b, key_a, idx_a),
                (18, key_a, idx_a, key_b, idx_b)):
            _radix_pass(shift, lane, ones, src_k, src_i, dst_k, dst_i, hist)
            if shift == 18:
                cnt_v[pl.ds(0, 16)] = hist[pl.ds(64 * 16, 16)]
            _radix_permute(shift, lane, src_k, src_i, dst_k, dst_i, hist)

        def sout_body(i, carry):
            k = key_b[pl.ds(i * 16, 16)]
            s = lax.bitcast_convert_type(0x3F800000 - k, jnp.float32)
            score_v[pl.ds(i * 16, 16)] = jnp.where(k < _INVALID_KEY, s, 0.0)
            return carry

        lax.fori_loop(0, _CHUNK, sout_body, 0, unroll=5)

        def gout_body(i, carry):
            x = idx_b[pl.ds(i * 16, 16)]
            key_a[pl.ds(i * 16, 16)] = plsc.load_gather(cls_v, [x])
            return carry

        lax.fori_loop(0, _CHUNK, gout_body, 0, unroll=5)

        pltpu.sync_copy(score_v, out_score.at[r])
        pltpu.sync_copy(key_a, out_cls.at[r])
        pltpu.sync_copy(cnt_v, out_cnt.at[r])

        for src_hbm, plane in ((b0_hbm, p0), (b1_hbm, p1),
                               (b2_hbm, p2), (b3_hbm, p3)):
            pltpu.sync_copy(src_hbm.at[r], plane)
        for plane, dst_hbm in ((p0, ob0), (p1, ob1), (p2, ob2), (p3, ob3)):
            def box_body(i, carry):
                x = idx_b[pl.ds(i * 16, 16)]
                pout[pl.ds(i * 16, 16)] = plsc.load_gather(plane, [x])
                return carry

            lax.fori_loop(0, _CHUNK, box_body, 0, unroll=5)
            pltpu.sync_copy(pout, dst_hbm.at[r])


def _sc_sort(scores, clss, planes):
    mesh = plsc.VectorSubcoreMesh(core_axis_name="c", subcore_axis_name="s",
                                  num_cores=2, num_subcores=16)
    return pl.kernel(
        _sc_sort_body,
        out_type=[
            jax.ShapeDtypeStruct((16, _N), jnp.float32),
            jax.ShapeDtypeStruct((16, _N), jnp.int32),
            jax.ShapeDtypeStruct((16, _N), jnp.float32),
            jax.ShapeDtypeStruct((16, _N), jnp.float32),
            jax.ShapeDtypeStruct((16, _N), jnp.float32),
            jax.ShapeDtypeStruct((16, _N), jnp.float32),
            jax.ShapeDtypeStruct((16, 16), jnp.int32),
        ],
        mesh=mesh,
        compiler_params=pltpu.CompilerParams(needs_layout_passes=False,
                                             use_tc_tiling_on_sc=False),
        scratch_types=[
            pltpu.VMEM((_N,), jnp.float32),
            pltpu.VMEM((_N,), jnp.int32),
            pltpu.VMEM((_N,), jnp.int32),
            pltpu.VMEM((_N,), jnp.int32),
            pltpu.VMEM((_N,), jnp.int32),
            pltpu.VMEM((_N,), jnp.int32),
            pltpu.VMEM((_BINS * 16,), jnp.int32),
            pltpu.VMEM((_N,), jnp.float32),
            pltpu.VMEM((_N,), jnp.float32),
            pltpu.VMEM((_N,), jnp.float32),
            pltpu.VMEM((_N,), jnp.float32),
            pltpu.VMEM((_N,), jnp.float32),
            pltpu.VMEM((16,), jnp.int32),
        ],
    )(scores, clss, *planes)


def kernel(feat_s8, feat_s16, feat_s32):
    scores, clss, *planes = _decode(feat_s8, feat_s16, feat_s32)
    out_score, out_cls, b0, b1, b2, b3, out_cnt = _sc_sort(scores, clss,
                                                           planes)
    out_boxes = jnp.stack([b0, b1, b2, b3], axis=2)
    return (out_boxes, out_score, out_cls.astype(jnp.int64), out_cnt[:, 0])


# speedup vs baseline: 1.2990x; 1.2266x over previous
import functools

import jax
import jax.numpy as jnp
from jax import lax
from jax.experimental import pallas as pl
from jax.experimental.pallas import tpu as pltpu
from jax.experimental.pallas import tpu_sc as plsc

_NUM_CLASSES = 80
_THRESH = 0.25
_STRIDES = (8.0, 16.0, 32.0)

_N = 8400
_LANES = 16
_CHUNK = _N // _LANES
_BINS = 512
_MASK = _BINS - 1
_INVALID_KEY = 1 << 24
_GCHUNKS = 66
_NPAD = _GCHUNKS * 128


def _decode_one(f, stride, w):
    idx = jax.lax.broadcasted_iota(jnp.int32, (1, f.shape[1]), 1)
    gx = (idx % w).astype(jnp.float32)
    gy = (idx // w).astype(jnp.float32)
    bx = (f[0:1] + gx) * stride
    by = (f[1:2] + gy) * stride
    bw = jnp.exp(f[2:3]) * stride
    bh = jnp.exp(f[3:4]) * stride
    x1 = bx - bw / 2.0
    y1 = by - bh / 2.0
    x2 = bx + bw / 2.0
    y2 = by + bh / 2.0
    obj = jax.nn.sigmoid(f[4:5])
    logits = f[5:5 + _NUM_CLASSES]
    lmax = jnp.max(logits, axis=0, keepdims=True)
    m = jax.nn.sigmoid(lmax) * obj
    ids = jax.lax.broadcasted_iota(jnp.int32, logits.shape, 0)
    cid = jnp.min(jnp.where(logits == lmax, ids, _NUM_CLASSES), axis=0,
                  keepdims=True)
    masked = jnp.where(m > _THRESH, m, -1.0)
    return masked, cid, x1, y1, x2, y2


def _decode_body(f8_ref, f16_ref, f32_ref, *out_refs):
    for lvl, (ref, stride, w) in enumerate(((f8_ref, 8.0, 80),
                                            (f16_ref, 16.0, 40),
                                            (f32_ref, 32.0, 20))):
        outs = _decode_one(ref[0], stride, w)
        for oix in range(6):
            out_refs[oix * 3 + lvl][0] = outs[oix]


def _decode(feat_s8, feat_s16, feat_s32):
    b = feat_s8.shape[0]
    fr = [f.reshape(b, 85, -1) for f in (feat_s8, feat_s16, feat_s32)]
    hws = (6400, 1600, 400)
    dts = (jnp.float32, jnp.int32) + (jnp.float32,) * 4
    outs = pl.pallas_call(
        _decode_body,
        grid=(b,),
        in_specs=[pl.BlockSpec((1, 85, hw), lambda i: (i, 0, 0))
                  for hw in hws],
        out_specs=[pl.BlockSpec((1, 1, hw), lambda i: (i, 0, 0))
                   for _ in dts for hw in hws],
        out_shape=[jax.ShapeDtypeStruct((b, 1, hw), dt)
                   for dt in dts for hw in hws],
    )(*fr)
    return [jnp.concatenate([outs[oix * 3 + lvl][:, 0]
                             for lvl in range(3)], axis=1)
            for oix in range(6)]


def _radix_pass(shift, lane, ones, key_src, idx_src, key_dst, idx_dst, hist):
    zeros = jnp.zeros((_LANES,), jnp.int32)

    def zero_body(b, carry):
        hist[pl.ds(b * 16, 16)] = zeros
        return carry

    lax.fori_loop(0, _BINS, zero_body, 0, unroll=8)

    def hist_body(i, carry):
        iv = lane * _CHUNK + i
        k = plsc.load_gather(key_src, [iv])
        d = (k >> shift) & _MASK
        plsc.addupdate_scatter(hist, [d * 16 + lane], ones)
        return carry

    lax.fori_loop(0, _CHUNK, hist_body, 0, unroll=5)

    def scan_body(b, carry):
        v = hist[pl.ds(b * 16, 16)]
        inc = plsc.cumsum(v)
        hist[pl.ds(b * 16, 16)] = inc - v + carry
        return carry + jnp.sum(v, axis=0)

    lax.fori_loop(0, _BINS, scan_body, jnp.int32(0), unroll=4)


def _radix_permute(shift, lane, key_src, idx_src, key_dst, idx_dst, hist):
    def perm_body(i, carry):
        iv = lane * _CHUNK + i
        k = plsc.load_gather(key_src, [iv])
        x = plsc.load_gather(idx_src, [iv])
        d = (k >> shift) & _MASK
        h = d * 16 + lane
        pos = plsc.load_gather(hist, [h])
        plsc.store_scatter(key_dst, [pos], k)
        plsc.store_scatter(idx_dst, [pos], x)
        plsc.store_scatter(hist, [h], pos + 1)
        return carry

    lax.fori_loop(0, _CHUNK, perm_body, 0, unroll=3)


def _sc_sort_body(score_hbm, cls_hbm, b0_hbm, b1_hbm, b2_hbm, b3_hbm,
                  out_score, out_cls, ob0, ob1, ob2, ob3, out_cnt,
                  score_v, cls_v, key_a, key_b, idx_a, idx_b, hist,
                  p0, p1, p2, p3, pout, cnt_v):
    nc = 2
    wid = lax.axis_index("s") * nc + lax.axis_index("c")
    lane = lax.iota(jnp.int32, 16)
    ones = jnp.ones((_LANES,), jnp.int32)

    @pl.when(wid < 16)
    def _():
        r = wid
        pltpu.sync_copy(score_hbm.at[r], score_v)
        pltpu.sync_copy(cls_hbm.at[r], cls_v)

        def init_body(i, carry):
            s = score_v[pl.ds(i * 16, 16)]
            b = lax.bitcast_convert_type(s, jnp.int32)
            k = jnp.where(s > 0.0, 0x3F800000 - b, _INVALID_KEY)
            key_a[pl.ds(i * 16, 16)] = k
            idx_a[pl.ds(i * 16, 16)] = lane + i * 16
            return carry

        lax.fori_loop(0, _CHUNK, init_body, 0, unroll=5)

        for shift, src_k, src_i, dst_k, dst_i in (
                (0, key_a, idx_a, key_b, idx_b),
                (9, key_b, idx_b, key_a, idx_a),
                (18, key_a, idx_a, key_b, idx_b)):
            _radix_pass(shift, lane, ones, src_k, src_i, dst_k, dst_i, hist)
            if shift == 18:
                cnt_v[pl.ds(0, 16)] = hist[pl.ds(64 * 16, 16)]
            _radix_permute(shift, lane, src_k, src_i, dst_k, dst_i, hist)

        def sout_body(i, carry):
            k = key_b[pl.ds(i * 16, 16)]
            s = lax.bitcast_convert_type(0x3F800000 - k, jnp.float32)
            score_v[pl.ds(i * 16, 16)] = jnp.where(k < _INVALID_KEY, s, 0.0)
            return carry

        lax.fori_loop(0, _CHUNK, sout_body, 0, unroll=5)

        def gout_body(i, carry):
            x = idx_b[pl.ds(i * 16, 16)]
            key_a[pl.ds(i * 16, 16)] = plsc.load_gather(cls_v, [x])
            return carry

        lax.fori_loop(0, _CHUNK, gout_body, 0, unroll=5)

        pltpu.sync_copy(score_v, out_score.at[r])
        pltpu.sync_copy(key_a, out_cls.at[r])
        pltpu.sync_copy(cnt_v, out_cnt.at[r])

        for src_hbm, plane in ((b0_hbm, p0), (b1_hbm, p1),
                               (b2_hbm, p2), (b3_hbm, p3)):
            pltpu.sync_copy(src_hbm.at[r], plane)
        for plane, dst_hbm in ((p0, ob0), (p1, ob1), (p2, ob2), (p3, ob3)):
            def box_body(i, carry):
                x = idx_b[pl.ds(i * 16, 16)]
                pout[pl.ds(i * 16, 16)] = plsc.load_gather(plane, [x])
                return carry

            lax.fori_loop(0, _CHUNK, box_body, 0, unroll=5)
            pltpu.sync_copy(pout, dst_hbm.at[r])


def _sc_sort(scores, clss, planes):
    mesh = plsc.VectorSubcoreMesh(core_axis_name="c", subcore_axis_name="s",
                                  num_cores=2, num_subcores=16)
    return pl.kernel(
        _sc_sort_body,
        out_type=[
            jax.ShapeDtypeStruct((16, _N), jnp.float32),
            jax.ShapeDtypeStruct((16, _N), jnp.int32),
            jax.ShapeDtypeStruct((16, _N), jnp.float32),
            jax.ShapeDtypeStruct((16, _N), jnp.float32),
            jax.ShapeDtypeStruct((16, _N), jnp.float32),
            jax.ShapeDtypeStruct((16, _N), jnp.float32),
            jax.ShapeDtypeStruct((16, 16), jnp.int32),
        ],
        mesh=mesh,
        compiler_params=pltpu.CompilerParams(needs_layout_passes=False,
                                             use_tc_tiling_on_sc=False),
        scratch_types=[
            pltpu.VMEM((_N,), jnp.float32),
            pltpu.VMEM((_N,), jnp.int32),
            pltpu.VMEM((_N,), jnp.int32),
            pltpu.VMEM((_N,), jnp.int32),
            pltpu.VMEM((_N,), jnp.int32),
            pltpu.VMEM((_N,), jnp.int32),
            pltpu.VMEM((_BINS * 16,), jnp.int32),
            pltpu.VMEM((_N,), jnp.float32),
            pltpu.VMEM((_N,), jnp.float32),
            pltpu.VMEM((_N,), jnp.float32),
            pltpu.VMEM((_N,), jnp.float32),
            pltpu.VMEM((_N,), jnp.float32),
            pltpu.VMEM((16,), jnp.int32),
        ],
    )(scores, clss, *planes)


def kernel(feat_s8, feat_s16, feat_s32):
    scores, clss, *planes = _decode(feat_s8, feat_s16, feat_s32)
    out_score, out_cls, b0, b1, b2, b3, out_cnt = _sc_sort(scores, clss,
                                                           planes)
    out_boxes = jnp.stack([b0, b1, b2, b3], axis=2)
    return (out_boxes, out_score, out_cls.astype(jnp.int64), out_cnt[:, 0])


# speedup vs baseline: 1.3947x; 1.0737x over previous
import functools

import jax
import jax.numpy as jnp
from jax import lax
from jax.experimental import pallas as pl
from jax.experimental.pallas import tpu as pltpu
from jax.experimental.pallas import tpu_sc as plsc

_NUM_CLASSES = 80
_THRESH = 0.25
_STRIDES = (8.0, 16.0, 32.0)

_N = 8400
_LANES = 16
_CHUNK = _N // _LANES
_BINS = 512
_MASK = _BINS - 1
_INVALID_KEY = 1 << 24
_GCHUNKS = 66
_NPAD = _GCHUNKS * 128


def _decode_one(f, stride, w):
    idx = jax.lax.broadcasted_iota(jnp.int32, (1, f.shape[1]), 1)
    gx = (idx % w).astype(jnp.float32)
    gy = (idx // w).astype(jnp.float32)
    bx = (f[0:1] + gx) * stride
    by = (f[1:2] + gy) * stride
    bw = jnp.exp(f[2:3]) * stride
    bh = jnp.exp(f[3:4]) * stride
    x1 = bx - bw / 2.0
    y1 = by - bh / 2.0
    x2 = bx + bw / 2.0
    y2 = by + bh / 2.0
    obj = jax.nn.sigmoid(f[4:5])
    logits = f[5:5 + _NUM_CLASSES]
    lmax = jnp.max(logits, axis=0, keepdims=True)
    m = jax.nn.sigmoid(lmax) * obj
    ids = jax.lax.broadcasted_iota(jnp.int32, logits.shape, 0)
    cid = jnp.min(jnp.where(logits == lmax, ids, _NUM_CLASSES), axis=0,
                  keepdims=True)
    masked = jnp.where(m > _THRESH, m, -1.0)
    return masked, cid, x1, y1, x2, y2


def _decode_body(f8_ref, f16_ref, f32_ref, *out_refs):
    for lvl, (ref, stride, w) in enumerate(((f8_ref, 8.0, 80),
                                            (f16_ref, 16.0, 40),
                                            (f32_ref, 32.0, 20))):
        outs = _decode_one(ref[0], stride, w)
        for oix in range(6):
            out_refs[oix * 3 + lvl][0] = outs[oix]


def _decode(feat_s8, feat_s16, feat_s32):
    b = feat_s8.shape[0]
    fr = [f.reshape(b, 85, -1) for f in (feat_s8, feat_s16, feat_s32)]
    hws = (6400, 1600, 400)
    dts = (jnp.float32, jnp.int32) + (jnp.float32,) * 4
    outs = pl.pallas_call(
        _decode_body,
        grid=(b,),
        in_specs=[pl.BlockSpec((1, 85, hw), lambda i: (i, 0, 0))
                  for hw in hws],
        out_specs=[pl.BlockSpec((1, 1, hw), lambda i: (i, 0, 0))
                   for _ in dts for hw in hws],
        out_shape=[jax.ShapeDtypeStruct((b, 1, hw), dt)
                   for dt in dts for hw in hws],
    )(*fr)
    return [jnp.concatenate([outs[oix * 3 + lvl][:, 0]
                             for lvl in range(3)], axis=1)
            for oix in range(6)]


def _radix_pass(shift, lane, ones, key_src, idx_src, key_dst, idx_dst, hist):
    zeros = jnp.zeros((_LANES,), jnp.int32)

    def zero_body(b, carry):
        hist[pl.ds(b * 16, 16)] = zeros
        return carry

    lax.fori_loop(0, _BINS, zero_body, 0, unroll=8)

    def hist_body(i, carry):
        iv = lane * _CHUNK + i
        k = plsc.load_gather(key_src, [iv])
        d = (k >> shift) & _MASK
        plsc.addupdate_scatter(hist, [d * 16 + lane], ones)
        return carry

    lax.fori_loop(0, _CHUNK, hist_body, 0, unroll=5)

    def scan_body(b, carry):
        v = hist[pl.ds(b * 16, 16)]
        inc = plsc.cumsum(v)
        hist[pl.ds(b * 16, 16)] = inc - v + carry
        return carry + jnp.sum(v, axis=0)

    lax.fori_loop(0, _BINS, scan_body, jnp.int32(0), unroll=4)


def _radix_permute(shift, lane, key_src, idx_src, key_dst, idx_dst, hist):
    def perm_body(i, carry):
        iv = lane * _CHUNK + i
        k = plsc.load_gather(key_src, [iv])
        x = plsc.load_gather(idx_src, [iv])
        d = (k >> shift) & _MASK
        h = d * 16 + lane
        pos = plsc.load_gather(hist, [h])
        plsc.store_scatter(key_dst, [pos], k)
        plsc.store_scatter(idx_dst, [pos], x)
        plsc.store_scatter(hist, [h], pos + 1)
        return carry

    lax.fori_loop(0, _CHUNK, perm_body, 0, unroll=3)


def _gather_plane(idx_b, plane, pout, dst_hbm, r):
    def box_body(i, carry):
        x = idx_b[pl.ds(i * 16, 16)]
        pout[pl.ds(i * 16, 16)] = plsc.load_gather(plane, [x])
        return carry

    lax.fori_loop(0, _CHUNK, box_body, 0, unroll=5)
    pltpu.sync_copy(pout, dst_hbm.at[r])


def _sc_sort_body(score_hbm, cls_hbm, b0_hbm, b1_hbm, b2_hbm, b3_hbm,
                  out_score, out_cls, ob0, ob1, ob2, ob3, out_cnt,
                  score_v, cls_v, key_a, key_b, idx_a, idx_b, hist,
                  p0, p1, p2, p3, pout, cnt_v, shared):
    nc = 2
    wid = lax.axis_index("s") * nc + lax.axis_index("c")
    lane = lax.iota(jnp.int32, 16)
    ones = jnp.ones((_LANES,), jnp.int32)

    @pl.when(wid < 16)
    def _():
        r = wid
        pltpu.sync_copy(score_hbm.at[r], score_v)
        pltpu.sync_copy(cls_hbm.at[r], cls_v)

        def init_body(i, carry):
            s = score_v[pl.ds(i * 16, 16)]
            b = lax.bitcast_convert_type(s, jnp.int32)
            k = jnp.where(s > 0.0, 0x3F800000 - b, _INVALID_KEY)
            key_a[pl.ds(i * 16, 16)] = k
            idx_a[pl.ds(i * 16, 16)] = lane + i * 16
            return carry

        lax.fori_loop(0, _CHUNK, init_body, 0, unroll=5)

        for shift, src_k, src_i, dst_k, dst_i in (
                (0, key_a, idx_a, key_b, idx_b),
                (9, key_b, idx_b, key_a, idx_a),
                (18, key_a, idx_a, key_b, idx_b)):
            _radix_pass(shift, lane, ones, src_k, src_i, dst_k, dst_i, hist)
            if shift == 18:
                cnt_v[pl.ds(0, 16)] = hist[pl.ds(64 * 16, 16)]
            _radix_permute(shift, lane, src_k, src_i, dst_k, dst_i, hist)

        pltpu.sync_copy(idx_b, shared.at[r, 0])
        pltpu.sync_copy(key_b, shared.at[r, 1])

    @pl.when(wid >= 16)
    def _():
        rr = wid - 16
        pltpu.sync_copy(b2_hbm.at[rr], p2)
        pltpu.sync_copy(b3_hbm.at[rr], p3)

    plsc.subcore_barrier()

    @pl.when(wid < 16)
    def _():
        r = wid

        def gout_body(i, carry):
            x = idx_b[pl.ds(i * 16, 16)]
            key_a[pl.ds(i * 16, 16)] = plsc.load_gather(cls_v, [x])
            return carry

        lax.fori_loop(0, _CHUNK, gout_body, 0, unroll=5)

        pltpu.sync_copy(key_a, out_cls.at[r])
        pltpu.sync_copy(cnt_v, out_cnt.at[r])

        for src_hbm, plane in ((b0_hbm, p0), (b1_hbm, p1)):
            pltpu.sync_copy(src_hbm.at[r], plane)
        _gather_plane(idx_b, p0, pout, ob0, r)
        _gather_plane(idx_b, p1, score_v, ob1, r)

    @pl.when(wid >= 16)
    def _():
        rr = wid - 16
        pltpu.sync_copy(shared.at[rr, 0], idx_b)
        pltpu.sync_copy(shared.at[rr, 1], key_b)

        def sout_body(i, carry):
            k = key_b[pl.ds(i * 16, 16)]
            s = lax.bitcast_convert_type(0x3F800000 - k, jnp.float32)
            score_v[pl.ds(i * 16, 16)] = jnp.where(k < _INVALID_KEY, s, 0.0)
            return carry

        lax.fori_loop(0, _CHUNK, sout_body, 0, unroll=5)
        pltpu.sync_copy(score_v, out_score.at[rr])
        _gather_plane(idx_b, p2, pout, ob2, rr)
        _gather_plane(idx_b, p3, score_v, ob3, rr)


def _sc_sort(scores, clss, planes):
    mesh = plsc.VectorSubcoreMesh(core_axis_name="c", subcore_axis_name="s",
                                  num_cores=2, num_subcores=16)
    return pl.kernel(
        _sc_sort_body,
        out_type=[
            jax.ShapeDtypeStruct((16, _N), jnp.float32),
            jax.ShapeDtypeStruct((16, _N), jnp.int32),
            jax.ShapeDtypeStruct((16, _N), jnp.float32),
            jax.ShapeDtypeStruct((16, _N), jnp.float32),
            jax.ShapeDtypeStruct((16, _N), jnp.float32),
            jax.ShapeDtypeStruct((16, _N), jnp.float32),
            jax.ShapeDtypeStruct((16, 16), jnp.int32),
        ],
        mesh=mesh,
        compiler_params=pltpu.CompilerParams(needs_layout_passes=False,
                                             use_tc_tiling_on_sc=False),
        scratch_types=[
            pltpu.VMEM((_N,), jnp.float32),
            pltpu.VMEM((_N,), jnp.int32),
            pltpu.VMEM((_N,), jnp.int32),
            pltpu.VMEM((_N,), jnp.int32),
            pltpu.VMEM((_N,), jnp.int32),
            pltpu.VMEM((_N,), jnp.int32),
            pltpu.VMEM((_BINS * 16,), jnp.int32),
            pltpu.VMEM((_N,), jnp.float32),
            pltpu.VMEM((_N,), jnp.float32),
            pltpu.VMEM((_N,), jnp.float32),
            pltpu.VMEM((_N,), jnp.float32),
            pltpu.VMEM((_N,), jnp.float32),
            pltpu.VMEM((16,), jnp.int32),
            pltpu.VMEM_SHARED((16, 2, _N), jnp.int32),
        ],
    )(scores, clss, *planes)


def kernel(feat_s8, feat_s16, feat_s32):
    scores, clss, *planes = _decode(feat_s8, feat_s16, feat_s32)
    out_score, out_cls, b0, b1, b2, b3, out_cnt = _sc_sort(scores, clss,
                                                           planes)
    out_boxes = jnp.stack([b0, b1, b2, b3], axis=2)
    return (out_boxes, out_score, out_cls.astype(jnp.int64), out_cnt[:, 0])
